# baseline (speedup 1.0000x reference)
import functools
import jax
import jax.numpy as jnp
from jax import lax
from jax.experimental import pallas as pl
from jax.experimental.pallas import tpu as pltpu
from jax.experimental.pallas import tpu_sc as plsc

N_NODES = 10000
F_IN = 128
HID = 128
NCLASS = 16

NC = 2
NS = 16
FH = HID // NC
CHUNK = 128
N_PAD = 10240
ROWS_PER_TILE = N_PAD // NS


def _mm_body(x_ref, w_ref, o_ref):
    r = jnp.dot(x_ref[...], w_ref[...], preferred_element_type=jnp.float32)
    o_ref[0] = r[:, :FH]
    o_ref[1] = r[:, FH:]


def _dense_matmul_split(x, w):
    n = x.shape[0]
    blk = 2000
    nb = n // blk
    out = pl.pallas_call(
        _mm_body,
        grid=(nb,),
        in_specs=[
            pl.BlockSpec((blk, F_IN), lambda i: (i, 0)),
            pl.BlockSpec((F_IN, HID), lambda i: (0, 0)),
        ],
        out_specs=pl.BlockSpec((NC, blk, FH), lambda i: (0, i, 0)),
        out_shape=jax.ShapeDtypeStruct((NC, n, FH), jnp.float32),
    )(x, w)
    return out.reshape(NC * n, FH)


def _spmm_body(h_hbm, src_hbm, dst_hbm, w_hbm, acc_out, c_out,
               src_v, dst_v, w_v, rows0, rows1, acc_sh, c_sh, g0, g1):
    cid = lax.axis_index("c")
    sid = lax.axis_index("s")
    n_chunks = dst_hbm.shape[1]
    n2 = n_chunks // 2

    def zero_row(r, _):
        for f in range(FH // 16):
            rows0[r, pl.ds(f * 16, 16)] = jnp.zeros((16,), jnp.float32)
        return _
    lax.fori_loop(0, CHUNK, zero_row, None)
    for t in range(ROWS_PER_TILE // CHUNK):
        off = sid * ROWS_PER_TILE + t * CHUNK
        pltpu.sync_copy(rows0, acc_sh.at[pl.ds(off, CHUNK)])
        for u in range(CHUNK // FH):
            pltpu.sync_copy(rows0.at[0], c_sh.at[pl.ds(off + u * FH, FH)])
    plsc.subcore_barrier()

    pltpu.sync_copy(src_hbm.at[cid, sid], src_v)
    pltpu.sync_copy(dst_hbm.at[sid], dst_v)
    pltpu.sync_copy(w_hbm.at[sid], w_v)

    def gather(j, buf, sem):
        return pltpu.make_async_copy(h_hbm.at[src_v.at[j]], buf, sem)

    def scale(buf, j):
        def scale_block(b, __):
            wvec = w_v[j, pl.ds(b * 16, 16)]
            for l in range(16):
                i = b * 16 + l
                wb = jnp.full((16,), wvec[l], jnp.float32)
                for f in range(FH // 16):
                    sl = pl.ds(f * 16, 16)
                    buf[i, sl] = buf[i, sl] * wb
            return __
        lax.fori_loop(0, CHUNK // 16, scale_block, None)

    def scatter_sync(buf, j):
        pltpu.sync_copy(buf, acc_sh.at[dst_v.at[j]], add=True)

        @pl.when(cid == 0)
        def _():
            pltpu.sync_copy(w_v.at[j], c_sh.at[src_v.at[j]], add=True)

    gather(0, rows0, g0).start()

    def pipe(j2, _):
        a = j2 * 2
        b = a + 1
        gather(a, rows0, g0).wait()
        gather(b, rows1, g1).start()
        scale(rows0, a)
        scatter_sync(rows0, a)
        gather(b, rows1, g1).wait()
        gather(a + 2, rows0, g0).start()
        scale(rows1, b)
        scatter_sync(rows1, b)
        return _

    lax.fori_loop(0, n2, pipe, None)
    gather(n_chunks, rows0, g0).wait()
    plsc.subcore_barrier()

    off = sid * ROWS_PER_TILE
    pltpu.sync_copy(acc_sh.at[pl.ds(off, ROWS_PER_TILE)],
                    acc_out.at[cid, pl.ds(off, ROWS_PER_TILE)])
    pltpu.sync_copy(c_sh.at[pl.ds(off, ROWS_PER_TILE)],
                    c_out.at[cid, pl.ds(off, ROWS_PER_TILE)])


def _spmm(h2, src4, dst3, w3):
    n_chunks = dst3.shape[1]
    kern = functools.partial(
        pl.kernel,
        out_type=(
            jax.ShapeDtypeStruct((NC, N_PAD, FH), jnp.float32),
            jax.ShapeDtypeStruct((NC, N_PAD), jnp.float32),
        ),
        mesh=plsc.VectorSubcoreMesh(core_axis_name="c", subcore_axis_name="s"),
        compiler_params=pltpu.CompilerParams(use_tc_tiling_on_sc=False),
        scratch_types=[
            pltpu.VMEM((n_chunks + 1, CHUNK), jnp.int32),
            pltpu.VMEM((n_chunks, CHUNK), jnp.int32),
            pltpu.VMEM((n_chunks, CHUNK), jnp.float32),
            pltpu.VMEM((CHUNK, FH), jnp.float32),
            pltpu.VMEM((CHUNK, FH), jnp.float32),
            pltpu.VMEM_SHARED((N_PAD, FH), jnp.float32),
            pltpu.VMEM_SHARED((N_PAD,), jnp.float32),
            pltpu.SemaphoreType.DMA,
            pltpu.SemaphoreType.DMA,
        ],
    )(_spmm_body)
    return kern(h2, src4, dst3, w3)


def _reduce_body(a0_ref, a1_ref, c0_ref, c1_ref, w2a_ref, w2b_ref, o_ref):
    i = pl.program_id(0)
    cb = c0_ref[...] + c1_ref[...]
    s0 = jnp.sum(jnp.maximum(a0_ref[...], 0.0) * cb, axis=0)[None, :]
    s1 = jnp.sum(jnp.maximum(a1_ref[...], 0.0) * cb, axis=0)[None, :]
    val = (jnp.dot(s0, w2a_ref[...], preferred_element_type=jnp.float32)
           + jnp.dot(s1, w2b_ref[...], preferred_element_type=jnp.float32)
           ) * (1.0 / N_NODES)

    @pl.when(i == 0)
    def _():
        o_ref[...] = val

    @pl.when(i > 0)
    def _():
        o_ref[...] = o_ref[...] + val


def _reduce(acc, c, w2):
    blk = 1024
    grid = N_PAD // blk
    return pl.pallas_call(
        _reduce_body,
        grid=(grid,),
        in_specs=[
            pl.BlockSpec((blk, FH), lambda i: (i, 0)),
            pl.BlockSpec((blk, FH), lambda i: (i, 0)),
            pl.BlockSpec((blk, 1), lambda i: (i, 0)),
            pl.BlockSpec((blk, 1), lambda i: (i, 0)),
            pl.BlockSpec((FH, NCLASS), lambda i: (0, 0)),
            pl.BlockSpec((FH, NCLASS), lambda i: (0, 0)),
        ],
        out_specs=pl.BlockSpec((1, NCLASS), lambda i: (0, 0)),
        out_shape=jax.ShapeDtypeStruct((1, NCLASS), jnp.float32),
    )(acc[0], acc[1], c[0].reshape(N_PAD, 1), c[1].reshape(N_PAD, 1),
      w2[:FH], w2[FH:])


def kernel(x, edge_index, edge_weight, W1, W2):
    n = x.shape[0]
    e = edge_weight.shape[0]
    per_tile = -(-e // (NS * 2 * CHUNK)) * 2 * CHUNK
    e_pad = per_tile * NS
    n_chunks = per_tile // CHUNK

    src = jnp.asarray(edge_index[0], jnp.int32)
    dst = jnp.asarray(edge_index[1], jnp.int32)
    w = jnp.asarray(edge_weight, jnp.float32)
    pad = e_pad - e
    src3 = jnp.pad(
        jnp.pad(src, (0, pad)).reshape(NS, n_chunks, CHUNK),
        ((0, 0), (0, 1), (0, 0)))
    dst3 = jnp.pad(dst, (0, pad)).reshape(NS, n_chunks, CHUNK)
    src4 = jnp.stack([src3, src3 + n])
    w3 = jnp.pad(w, (0, pad)).reshape(NS, n_chunks, CHUNK)

    h2 = _dense_matmul_split(x, W1)
    acc, c = _spmm(h2, src4, dst3, w3)
    return _reduce(acc, c, W2)


# speedup vs baseline: 1.0717x; 1.0717x over previous
import functools
import jax
import jax.numpy as jnp
from jax import lax
from jax.experimental import pallas as pl
from jax.experimental.pallas import tpu as pltpu
from jax.experimental.pallas import tpu_sc as plsc

N_NODES = 10000
F_IN = 128
HID = 128
NCLASS = 16

NC = 2
NS = 16
FH = HID // NC
CHUNK = 128
N_PAD = 10240
ROWS_PER_TILE = N_PAD // NS


def _mm_body(x_ref, w_ref, o_ref):
    r = jnp.dot(x_ref[...], w_ref[...], preferred_element_type=jnp.float32)
    o_ref[0] = r[:, :FH]
    o_ref[1] = r[:, FH:]


def _dense_matmul_split(x, w):
    n = x.shape[0]
    blk = 2000
    nb = n // blk
    out = pl.pallas_call(
        _mm_body,
        grid=(nb,),
        in_specs=[
            pl.BlockSpec((blk, F_IN), lambda i: (i, 0)),
            pl.BlockSpec((F_IN, HID), lambda i: (0, 0)),
        ],
        out_specs=pl.BlockSpec((NC, blk, FH), lambda i: (0, i, 0)),
        out_shape=jax.ShapeDtypeStruct((NC, n, FH), jnp.float32),
    )(x, w)
    return out.reshape(NC * n, FH)


def _spmm_body(h_hbm, src_hbm, dst_hbm, w_hbm, acc_out, c_out,
               src_v, dst_v, w_v, rows0, rows1, rows2, acc_sh, c_sh,
               g0, g1, g2, s0, s1, s2, csem):
    cid = lax.axis_index("c")
    sid = lax.axis_index("s")
    n_chunks = dst_hbm.shape[1]
    rows = (rows0, rows1, rows2)
    gsem = (g0, g1, g2)
    ssem = (s0, s1, s2)

    def zero_rows(buf):
        def zero_row(r, _):
            for f in range(FH // 16):
                buf[r, pl.ds(f * 16, 16)] = jnp.zeros((16,), jnp.float32)
            return _
        lax.fori_loop(0, CHUNK, zero_row, None)
    zero_rows(rows0)
    zero_rows(rows1)
    zero_rows(rows2)
    for t in range(ROWS_PER_TILE // CHUNK):
        off = sid * ROWS_PER_TILE + t * CHUNK
        pltpu.sync_copy(rows0, acc_sh.at[pl.ds(off, CHUNK)])
        for u in range(CHUNK // FH):
            pltpu.sync_copy(rows0.at[0], c_sh.at[pl.ds(off + u * FH, FH)])
    plsc.subcore_barrier()

    pltpu.sync_copy(src_hbm.at[cid, sid], src_v)
    pltpu.sync_copy(dst_hbm.at[sid], dst_v)
    pltpu.sync_copy(w_hbm.at[sid], w_v)

    def gather(j, buf, sem):
        return pltpu.make_async_copy(h_hbm.at[src_v.at[j]], buf, sem)

    def scatter(j, buf, sem):
        return pltpu.make_async_copy(buf, acc_sh.at[dst_v.at[j]], sem)

    def cscat(j):
        return pltpu.make_async_copy(w_v.at[j], c_sh.at[src_v.at[j]], csem)

    def scale(buf, j):
        def scale_block(b, __):
            wvec = w_v[j, pl.ds(b * 16, 16)]
            for l in range(16):
                i = b * 16 + l
                wb = jnp.full((16,), wvec[l], jnp.float32)
                for f in range(FH // 16):
                    sl = pl.ds(f * 16, 16)
                    buf[i, sl] = buf[i, sl] * wb
            return __
        lax.fori_loop(0, CHUNK // 16, scale_block, None)

    n3 = n_chunks // 3

    scatter(0, rows1, s1).start(add=True)
    scatter(0, rows2, s2).start(add=True)
    gather(0, rows0, g0).start()

    @pl.when(cid == 0)
    def _():
        cscat(0).start(add=True)

    def step(x, p):
        pn = (p + 1) % 3
        gather(0, rows[p], gsem[p]).wait()
        scatter(0, rows[pn], ssem[pn]).wait()
        gather(x + 1, rows[pn], gsem[pn]).start()
        scale(rows[p], x)
        scatter(x, rows[p], ssem[p]).start(add=True)

        @pl.when(cid == 0)
        def _():
            cscat(0).wait()
            cscat(x + 1).start(add=True)

    def pipe(j3, _):
        for q in range(3):
            step(j3 * 3 + q, q)
        return _

    lax.fori_loop(0, n3, pipe, None)
    gather(0, rows[0], gsem[0]).wait()
    scatter(0, rows[1], ssem[1]).wait()
    scatter(0, rows[2], ssem[2]).wait()

    @pl.when(cid == 0)
    def _():
        cscat(0).wait()
    plsc.subcore_barrier()

    off = sid * ROWS_PER_TILE
    pltpu.sync_copy(acc_sh.at[pl.ds(off, ROWS_PER_TILE)],
                    acc_out.at[cid, pl.ds(off, ROWS_PER_TILE)])
    pltpu.sync_copy(c_sh.at[pl.ds(off, ROWS_PER_TILE)],
                    c_out.at[cid, pl.ds(off, ROWS_PER_TILE)])


def _spmm(h2, src4, dst3, w3):
    n_chunks = dst3.shape[1]
    kern = functools.partial(
        pl.kernel,
        out_type=(
            jax.ShapeDtypeStruct((NC, N_PAD, FH), jnp.float32),
            jax.ShapeDtypeStruct((NC, N_PAD), jnp.float32),
        ),
        mesh=plsc.VectorSubcoreMesh(core_axis_name="c", subcore_axis_name="s"),
        compiler_params=pltpu.CompilerParams(use_tc_tiling_on_sc=False),
        scratch_types=[
            pltpu.VMEM((n_chunks + 1, CHUNK), jnp.int32),
            pltpu.VMEM((n_chunks, CHUNK), jnp.int32),
            pltpu.VMEM((n_chunks + 1, CHUNK), jnp.float32),
            pltpu.VMEM((CHUNK, FH), jnp.float32),
            pltpu.VMEM((CHUNK, FH), jnp.float32),
            pltpu.VMEM((CHUNK, FH), jnp.float32),
            pltpu.VMEM_SHARED((N_PAD, FH), jnp.float32),
            pltpu.VMEM_SHARED((N_PAD,), jnp.float32),
            pltpu.SemaphoreType.DMA,
            pltpu.SemaphoreType.DMA,
            pltpu.SemaphoreType.DMA,
            pltpu.SemaphoreType.DMA,
            pltpu.SemaphoreType.DMA,
            pltpu.SemaphoreType.DMA,
            pltpu.SemaphoreType.DMA,
        ],
    )(_spmm_body)
    return kern(h2, src4, dst3, w3)


def _reduce_body(a0_ref, a1_ref, c0_ref, c1_ref, w2a_ref, w2b_ref, o_ref):
    i = pl.program_id(0)
    cb = c0_ref[...] + c1_ref[...]
    s0 = jnp.sum(jnp.maximum(a0_ref[...], 0.0) * cb, axis=0)[None, :]
    s1 = jnp.sum(jnp.maximum(a1_ref[...], 0.0) * cb, axis=0)[None, :]
    val = (jnp.dot(s0, w2a_ref[...], preferred_element_type=jnp.float32)
           + jnp.dot(s1, w2b_ref[...], preferred_element_type=jnp.float32)
           ) * (1.0 / N_NODES)

    @pl.when(i == 0)
    def _():
        o_ref[...] = val

    @pl.when(i > 0)
    def _():
        o_ref[...] = o_ref[...] + val


def _reduce(acc, c, w2):
    blk = 1024
    grid = N_PAD // blk
    return pl.pallas_call(
        _reduce_body,
        grid=(grid,),
        in_specs=[
            pl.BlockSpec((blk, FH), lambda i: (i, 0)),
            pl.BlockSpec((blk, FH), lambda i: (i, 0)),
            pl.BlockSpec((blk, 1), lambda i: (i, 0)),
            pl.BlockSpec((blk, 1), lambda i: (i, 0)),
            pl.BlockSpec((FH, NCLASS), lambda i: (0, 0)),
            pl.BlockSpec((FH, NCLASS), lambda i: (0, 0)),
        ],
        out_specs=pl.BlockSpec((1, NCLASS), lambda i: (0, 0)),
        out_shape=jax.ShapeDtypeStruct((1, NCLASS), jnp.float32),
    )(acc[0], acc[1], c[0].reshape(N_PAD, 1), c[1].reshape(N_PAD, 1),
      w2[:FH], w2[FH:])


def kernel(x, edge_index, edge_weight, W1, W2):
    n = x.shape[0]
    e = edge_weight.shape[0]
    per_tile = -(-e // (NS * 3 * CHUNK)) * 3 * CHUNK
    e_pad = per_tile * NS
    n_chunks = per_tile // CHUNK

    src = jnp.asarray(edge_index[0], jnp.int32)
    dst = jnp.asarray(edge_index[1], jnp.int32)
    w = jnp.asarray(edge_weight, jnp.float32)
    pad = e_pad - e
    src3 = jnp.pad(
        jnp.pad(src, (0, pad)).reshape(NS, n_chunks, CHUNK),
        ((0, 0), (0, 1), (0, 0)))
    dst3 = jnp.pad(dst, (0, pad)).reshape(NS, n_chunks, CHUNK)
    src4 = jnp.stack([src3, src3 + n])
    w3 = jnp.pad(
        jnp.pad(w, (0, pad)).reshape(NS, n_chunks, CHUNK),
        ((0, 0), (0, 1), (0, 0)))

    h2 = _dense_matmul_split(x, W1)
    acc, c = _spmm(h2, src4, dst3, w3)
    return _reduce(acc, c, W2)
